# Initial kernel scaffold; baseline (speedup 1.0000x reference)
import jax
import jax.numpy as jnp
from jax.experimental import pallas as pl


def kernel(idx, targets, table):
    raise NotImplementedError("write your pallas kernel here")


# speedup vs baseline: 1.6696x; 1.6696x over previous
import functools

import jax
import jax.numpy as jnp
from jax import lax
from jax.experimental import pallas as pl
from jax.experimental.pallas import tpu as pltpu
from jax.experimental.pallas import tpu_sc as plsc

NC = 2
NS = 16
NW = NC * NS
L = 16

V = 1000
N_ROWS = 51200
B_PER_W = N_ROWS // NW
CHUNK = 64
NCHUNK = B_PER_W // CHUNK


def _lse_body(tab_ref, out_ref):
    x = tab_ref[...]
    m = jnp.max(x, axis=1, keepdims=True)
    s = jnp.sum(jnp.exp(x - m), axis=1, keepdims=True)
    out_ref[...] = m + jnp.log(s)


def _row_lse(table):
    out = pl.pallas_call(
        _lse_body,
        out_shape=jax.ShapeDtypeStruct((V, 1), jnp.float32),
    )(table)
    return out.reshape(V)


def _sc_body(table_hbm, idx_hbm, tgt_hbm, lse_hbm,
             out_hbm, part_hbm,
             idx_v, tgt_v, lse_v, buf, part_v, sem):
    c = lax.axis_index("c")
    s = lax.axis_index("s")
    wid = s * NC + c
    base = wid * B_PER_W

    pltpu.sync_copy(idx_hbm.at[pl.ds(base, B_PER_W)], idx_v)
    pltpu.sync_copy(tgt_hbm.at[pl.ds(base, B_PER_W)], tgt_v)
    pltpu.sync_copy(lse_hbm, lse_v)

    lane = lax.iota(jnp.int32, L)

    def chunk_body(g, acc):
        off = g * CHUNK
        pltpu.async_copy(table_hbm.at[idx_v.at[pl.ds(off, CHUNK)]],
                         buf, sem).wait()
        for j in range(CHUNK // L):
            o2 = off + j * L
            rows = lane + j * L
            t16 = tgt_v[pl.ds(o2, L)]
            i16 = idx_v[pl.ds(o2, L)]
            elem = plsc.load_gather(buf, [rows, t16])
            lseg = plsc.load_gather(lse_v, [i16])
            acc = acc + (lseg - elem)
        pltpu.sync_copy(buf, out_hbm.at[pl.ds(base + off, CHUNK)])
        return acc

    acc = lax.fori_loop(0, NCHUNK, chunk_body, jnp.zeros((L,), jnp.float32))
    part_v[...] = acc
    pltpu.sync_copy(part_v, part_hbm.at[wid])


_sc_gather = functools.partial(
    pl.kernel,
    out_type=(
        jax.ShapeDtypeStruct((N_ROWS, V), jnp.float32),
        jax.ShapeDtypeStruct((NW, L), jnp.float32),
    ),
    mesh=plsc.VectorSubcoreMesh(
        core_axis_name="c", subcore_axis_name="s",
        num_cores=NC, num_subcores=NS),
    scratch_types=[
        pltpu.VMEM((B_PER_W,), jnp.int32),
        pltpu.VMEM((B_PER_W,), jnp.int32),
        pltpu.VMEM((V,), jnp.float32),
        pltpu.VMEM((CHUNK, V), jnp.float32),
        pltpu.VMEM((L,), jnp.float32),
        pltpu.SemaphoreType.DMA,
    ],
    compiler_params=pltpu.CompilerParams(
        use_tc_tiling_on_sc=False, needs_layout_passes=False),
)(_sc_body)


def kernel(idx, targets, table):
    idx_f = idx.reshape(-1).astype(jnp.int32)
    tgt_f = targets.reshape(-1).astype(jnp.int32)
    lse = _row_lse(table)
    logits, parts = _sc_gather(table, idx_f, tgt_f, lse)
    loss = parts.sum() / jnp.float32(N_ROWS)
    return logits, loss


# speedup vs baseline: 1.6924x; 1.0137x over previous
import functools

import jax
import jax.numpy as jnp
from jax import lax
from jax.experimental import pallas as pl
from jax.experimental.pallas import tpu as pltpu
from jax.experimental.pallas import tpu_sc as plsc

NC = 2
NS = 16
NW = NC * NS
L = 16

V = 1000
N_ROWS = 51200
B_PER_W = N_ROWS // NW
CHUNK = 32
NCHUNK = B_PER_W // CHUNK
NPAIR = NCHUNK // 2


def _lse_body(tab_ref, out_ref):
    x = tab_ref[...]
    m = jnp.max(x, axis=1, keepdims=True)
    s = jnp.sum(jnp.exp(x - m), axis=1, keepdims=True)
    out_ref[...] = m + jnp.log(s)


def _row_lse(table):
    out = pl.pallas_call(
        _lse_body,
        out_shape=jax.ShapeDtypeStruct((V, 1), jnp.float32),
    )(table)
    return out.reshape(V)


def _sc_body(table_hbm, idx_hbm, tgt_hbm, lse_hbm,
             out_hbm, part_hbm,
             idx_v, tgt_v, lse_v, buf0, buf1, part_v,
             g0, g1, s0, s1):
    c = lax.axis_index("c")
    s = lax.axis_index("s")
    wid = s * NC + c
    base = wid * B_PER_W

    pltpu.sync_copy(idx_hbm.at[pl.ds(base, B_PER_W)], idx_v)
    pltpu.sync_copy(tgt_hbm.at[pl.ds(base, B_PER_W)], tgt_v)
    pltpu.sync_copy(lse_hbm, lse_v)

    lane = lax.iota(jnp.int32, L)

    def gather_start(chunk, buf, sem):
        pltpu.async_copy(
            table_hbm.at[idx_v.at[pl.ds(chunk * CHUNK, CHUNK)]], buf, sem)

    def gather_wait(chunk, buf, sem):
        pltpu.make_async_copy(
            table_hbm.at[idx_v.at[pl.ds(chunk * CHUNK, CHUNK)]], buf,
            sem).wait()

    def scatter_start(chunk, buf, sem):
        pltpu.async_copy(
            buf, out_hbm.at[pl.ds(base + chunk * CHUNK, CHUNK)], sem)

    def scatter_wait(chunk, buf, sem):
        pltpu.make_async_copy(
            buf, out_hbm.at[pl.ds(base + chunk * CHUNK, CHUNK)],
            sem).wait()

    def loss(chunk, buf, acc):
        off = chunk * CHUNK
        for j in range(CHUNK // L):
            o2 = off + j * L
            rows = lane + j * L
            t16 = tgt_v[pl.ds(o2, L)]
            i16 = idx_v[pl.ds(o2, L)]
            elem = plsc.load_gather(buf, [rows, t16])
            lseg = plsc.load_gather(lse_v, [i16])
            acc = acc + (lseg - elem)
        return acc

    gather_start(0, buf0, g0)

    def pair_body(k, acc):
        c0 = 2 * k
        gather_wait(c0, buf0, g0)
        scatter_start(c0, buf0, s0)

        @pl.when(k > 0)
        def _():
            scatter_wait(c0 - 1, buf1, s1)

        gather_start(c0 + 1, buf1, g1)
        acc = loss(c0, buf0, acc)

        gather_wait(c0 + 1, buf1, g1)
        scatter_start(c0 + 1, buf1, s1)
        scatter_wait(c0, buf0, s0)

        @pl.when(k < NPAIR - 1)
        def _():
            gather_start(c0 + 2, buf0, g0)

        acc = loss(c0 + 1, buf1, acc)
        return acc

    acc = lax.fori_loop(0, NPAIR, pair_body, jnp.zeros((L,), jnp.float32))
    scatter_wait(NCHUNK - 1, buf1, s1)

    part_v[...] = acc
    pltpu.sync_copy(part_v, part_hbm.at[wid])


_sc_gather = functools.partial(
    pl.kernel,
    out_type=(
        jax.ShapeDtypeStruct((N_ROWS, V), jnp.float32),
        jax.ShapeDtypeStruct((NW, L), jnp.float32),
    ),
    mesh=plsc.VectorSubcoreMesh(
        core_axis_name="c", subcore_axis_name="s",
        num_cores=NC, num_subcores=NS),
    scratch_types=[
        pltpu.VMEM((B_PER_W,), jnp.int32),
        pltpu.VMEM((B_PER_W,), jnp.int32),
        pltpu.VMEM((V,), jnp.float32),
        pltpu.VMEM((CHUNK, V), jnp.float32),
        pltpu.VMEM((CHUNK, V), jnp.float32),
        pltpu.VMEM((L,), jnp.float32),
        pltpu.SemaphoreType.DMA,
        pltpu.SemaphoreType.DMA,
        pltpu.SemaphoreType.DMA,
        pltpu.SemaphoreType.DMA,
    ],
    compiler_params=pltpu.CompilerParams(
        use_tc_tiling_on_sc=False, needs_layout_passes=False),
)(_sc_body)


def kernel(idx, targets, table):
    idx_f = idx.reshape(-1).astype(jnp.int32)
    tgt_f = targets.reshape(-1).astype(jnp.int32)
    lse = _row_lse(table)
    logits, parts = _sc_gather(table, idx_f, tgt_f, lse)
    loss = parts.sum() / jnp.float32(N_ROWS)
    return logits, loss
